# baseline (speedup 1.0000x reference)
import functools

import jax
import jax.numpy as jnp
from jax.experimental import pallas as pl
from jax.experimental.pallas import tpu as pltpu

B, H, DIM, RANK = 64, 8, 256, 64
EPS = 1e-06
LANES = 2 * RANK


def _sasaki_kernel(ns_ref, gamma_ref, u_ref, v_ref, k_ref, vv_ref,
                   y_ref, uo_ref, vo_ref):
    bi = pl.program_id(0)
    hi = pl.program_id(1)
    j = ns_ref[bi, hi]
    g = jnp.clip(gamma_ref[bi, hi], 0.0, 1.0)

    A = u_ref[0, 0]
    K = k_ref[0, 0]

    t = jax.lax.dot_general(A, K, (((0,), (0,)), ((), ())),
                            preferred_element_type=jnp.float32)
    t0 = t[:, 0:1]
    t1 = t[:, 1:2]
    ridx = jax.lax.broadcasted_iota(jnp.int32, (LANES, 1), 0)
    even = (ridx % 2) == 0
    c = jnp.where(even, t0 + jnp.roll(t1, -1, axis=0),
                  jnp.roll(t1, 1, axis=0) - t0)

    alt = jnp.where(even, 1.0, -1.0)

    def pack(cv):
        swap = jnp.where(even, jnp.roll(cv, -1, axis=0), jnp.roll(cv, 1, axis=0))
        return jnp.concatenate([cv * alt, swap], axis=1)

    k_proj = jnp.dot(A, pack(c), preferred_element_type=jnp.float32)
    k_perp = K - k_proj
    sq = jnp.sum(k_perp * k_perp)
    norm = jnp.sqrt(jnp.maximum(sq, EPS * EPS))
    u_new = k_perp / norm

    li = jax.lax.broadcasted_iota(jnp.int32, (DIM, LANES), 1)
    mask_r = li == 2 * j
    mask_i = li == 2 * j + 1
    uo_ref[0, 0] = jnp.where(mask_r, u_new[:, 0:1],
                             jnp.where(mask_i, u_new[:, 1:2], A))

    ur = u_new[:, 0:1]
    ui = u_new[:, 1:2]
    kr = K[:, 0:1]
    ki = K[:, 1:2]
    dr = jnp.sum(ur * kr + ui * ki)
    di = jnp.sum(ur * ki - ui * kr)
    cq = jnp.where(ridx == 2 * j, dr, jnp.where(ridx == 2 * j + 1, di, c))

    Vb = v_ref[0, 0]
    vv = vv_ref[0, 0]
    V_new = g * jnp.where(mask_r, vv[:, 0:1],
                          jnp.where(mask_i, vv[:, 1:2], Vb))
    vo_ref[0, 0] = V_new
    y_ref[0, 0] = jnp.dot(V_new, pack(cq), preferred_element_type=jnp.float32)


@functools.partial(jax.jit, static_argnames=())
def kernel(U, V, k, v, gamma, next_slot, filled):
    U2 = U.reshape(B, H, DIM, LANES)
    V2 = V.reshape(B, H, DIM, LANES)

    big_spec = pl.BlockSpec((1, 1, DIM, LANES), lambda b, h: (b, h, 0, 0))
    vec_spec = pl.BlockSpec((1, 1, DIM, 2), lambda b, h: (b, h, 0, 0))
    smem_spec = pl.BlockSpec(memory_space=pltpu.SMEM)

    y, U_new2, V_new2 = pl.pallas_call(
        _sasaki_kernel,
        grid=(B, H),
        in_specs=[smem_spec, smem_spec, big_spec, big_spec, vec_spec, vec_spec],
        out_specs=[vec_spec, big_spec, big_spec],
        out_shape=[
            jax.ShapeDtypeStruct((B, H, DIM, 2), jnp.float32),
            jax.ShapeDtypeStruct((B, H, DIM, LANES), jnp.float32),
            jax.ShapeDtypeStruct((B, H, DIM, LANES), jnp.float32),
        ],
    )(next_slot, gamma, U2, V2, k, v)

    U_new = U_new2.reshape(B, H, DIM, RANK, 2)
    V_new = V_new2.reshape(B, H, DIM, RANK, 2)
    next_slot_new = (next_slot + 1) % RANK
    filled_new = jnp.minimum(filled + 1, jnp.full_like(filled, RANK))
    return (y, U_new, V_new, next_slot_new, filled_new)


# speedup vs baseline: 1.2505x; 1.2505x over previous
import functools

import jax
import jax.numpy as jnp
from jax.experimental import pallas as pl
from jax.experimental.pallas import tpu as pltpu

B, H, DIM, RANK = 64, 8, 256, 64
EPS = 1e-06
LANES = 2 * RANK


def _sasaki_kernel(ns_ref, gamma_ref, u_ref, v_ref, k_ref, vv_ref,
                   y_ref, uo_ref, vo_ref):
    bi = pl.program_id(0)

    ridx = jax.lax.broadcasted_iota(jnp.int32, (LANES, 1), 0)
    even = (ridx % 2) == 0
    alt = jnp.where(even, 1.0, -1.0)
    li = jax.lax.broadcasted_iota(jnp.int32, (DIM, LANES), 1)

    def pack(cv):
        swap = jnp.where(even, jnp.roll(cv, -1, axis=0), jnp.roll(cv, 1, axis=0))
        return jnp.concatenate([cv * alt, swap], axis=1)

    for h in range(H):
        j = ns_ref[bi, h]
        g = jnp.clip(gamma_ref[bi, h], 0.0, 1.0)

        A = u_ref[0, h]
        K = k_ref[0, h]

        t = jax.lax.dot_general(A, K, (((0,), (0,)), ((), ())),
                                preferred_element_type=jnp.float32)
        t0 = t[:, 0:1]
        t1 = t[:, 1:2]
        c = jnp.where(even, t0 + jnp.roll(t1, -1, axis=0),
                      jnp.roll(t1, 1, axis=0) - t0)

        k_proj = jnp.dot(A, pack(c), preferred_element_type=jnp.float32)
        k_perp = K - k_proj
        sq = jnp.sum(k_perp * k_perp)
        norm = jnp.sqrt(jnp.maximum(sq, EPS * EPS))
        u_new = k_perp / norm

        mask_r = li == 2 * j
        mask_i = li == 2 * j + 1
        uo_ref[0, h] = jnp.where(mask_r, u_new[:, 0:1],
                                 jnp.where(mask_i, u_new[:, 1:2], A))

        ur = u_new[:, 0:1]
        ui = u_new[:, 1:2]
        kr = K[:, 0:1]
        ki = K[:, 1:2]
        dr = jnp.sum(ur * kr + ui * ki)
        di = jnp.sum(ur * ki - ui * kr)
        cq = jnp.where(ridx == 2 * j, dr, jnp.where(ridx == 2 * j + 1, di, c))

        Vb = v_ref[0, h]
        vv = vv_ref[0, h]
        V_new = g * jnp.where(mask_r, vv[:, 0:1],
                              jnp.where(mask_i, vv[:, 1:2], Vb))
        vo_ref[0, h] = V_new
        y_ref[0, h] = jnp.dot(V_new, pack(cq),
                              preferred_element_type=jnp.float32)


@functools.partial(jax.jit, static_argnames=())
def kernel(U, V, k, v, gamma, next_slot, filled):
    U2 = U.reshape(B, H, DIM, LANES)
    V2 = V.reshape(B, H, DIM, LANES)

    big_spec = pl.BlockSpec((1, H, DIM, LANES), lambda b: (b, 0, 0, 0))
    vec_spec = pl.BlockSpec((1, H, DIM, 2), lambda b: (b, 0, 0, 0))
    smem_spec = pl.BlockSpec(memory_space=pltpu.SMEM)

    y, U_new2, V_new2 = pl.pallas_call(
        _sasaki_kernel,
        grid=(B,),
        in_specs=[smem_spec, smem_spec, big_spec, big_spec, vec_spec, vec_spec],
        out_specs=[vec_spec, big_spec, big_spec],
        out_shape=[
            jax.ShapeDtypeStruct((B, H, DIM, 2), jnp.float32),
            jax.ShapeDtypeStruct((B, H, DIM, LANES), jnp.float32),
            jax.ShapeDtypeStruct((B, H, DIM, LANES), jnp.float32),
        ],
    )(next_slot, gamma, U2, V2, k, v)

    U_new = U_new2.reshape(B, H, DIM, RANK, 2)
    V_new = V_new2.reshape(B, H, DIM, RANK, 2)
    next_slot_new = (next_slot + 1) % RANK
    filled_new = jnp.minimum(filled + 1, jnp.full_like(filled, RANK))
    return (y, U_new, V_new, next_slot_new, filled_new)


# speedup vs baseline: 2.3276x; 1.8613x over previous
import functools

import jax
import jax.numpy as jnp
from jax.experimental import pallas as pl
from jax.experimental.pallas import tpu as pltpu

B, H, DIM, RANK = 64, 8, 256, 64
EPS = 1e-06
QROWS = 4 * RANK
HALF = DIM // 2


def _to_q(x):
    return (x.reshape(B, H, 2, HALF, RANK, 2)
             .transpose(0, 1, 4, 2, 5, 3)
             .reshape(B, H, QROWS, HALF))


def _from_q(xq):
    return (xq.reshape(B, H, RANK, 2, 2, HALF)
              .transpose(0, 1, 3, 5, 2, 4)
              .reshape(B, H, DIM, RANK, 2))


def _to_q_vec(x):
    return (x.reshape(B, H, 2, HALF, 2)
             .transpose(0, 1, 2, 4, 3)
             .reshape(B, H, 4, HALF))


def _from_q_vec(xq):
    return (xq.reshape(B, H, 2, 2, HALF)
              .transpose(0, 1, 2, 4, 3)
              .reshape(B, H, DIM, 2))


def _sasaki_kernel(ns_ref, gamma_ref, u_ref, v_ref, kq_ref, vq_ref,
                   yq_ref, uo_ref, vo_ref):
    bi = pl.program_id(0)
    f32 = jnp.float32

    si = jax.lax.broadcasted_iota(jnp.int32, (QROWS, 1), 0)
    si4 = si // 4
    liq = jax.lax.broadcasted_iota(jnp.int32, (1, QROWS), 1) % 4
    e0 = (liq == 0).astype(f32)
    e1 = (liq == 1).astype(f32)
    e2 = (liq == 2).astype(f32)
    e3 = (liq == 3).astype(f32)
    t4r = jax.lax.broadcasted_iota(jnp.int32, (QROWS, 4), 0) % 4
    t4c = jax.lax.broadcasted_iota(jnp.int32, (QROWS, 4), 1)
    T4 = (t4r == t4c).astype(f32)
    s4r = jax.lax.broadcasted_iota(jnp.int32, (QROWS, QROWS), 0) // 4
    s4c = jax.lax.broadcasted_iota(jnp.int32, (QROWS, QROWS), 1) // 4
    S4 = (s4r == s4c).astype(f32)
    ones1 = jnp.ones((HALF, 1), f32)

    def mm(a, b):
        return jax.lax.dot_general(a, b, (((1,), (0,)), ((), ())),
                                   preferred_element_type=f32)

    def wrows(crT, ciT):
        w0 = crT * e0 - ciT * e1
        w1 = ciT * e0 + crT * e1
        w2 = crT * e2 - ciT * e3
        w3 = ciT * e2 + crT * e3
        return jnp.concatenate([w0, w1, w2, w3], axis=0)

    for h in range(H):
        j = ns_ref[bi, h]
        g = jnp.clip(gamma_ref[bi, h], 0.0, 1.0)

        A = u_ref[0, h]
        kq = kq_ref[0, h]
        kqs = jnp.concatenate([kq[1:2], -kq[0:1], kq[3:4], -kq[2:3]], axis=0)

        G1 = mm(T4, kq)
        G2 = mm(T4, kqs)
        rd1 = mm(A * G1, ones1)
        rd2 = mm(A * G2, ones1)
        cc = mm(S4, jnp.concatenate([rd1, rd2], axis=1))
        crb = cc[:, 0:1]
        cib = cc[:, 1:2]

        ct = jax.lax.transpose(cc, (1, 0))
        KP = mm(wrows(ct[0:1], ct[1:2]), A)
        k_perp = kq - KP
        sq = jnp.sum(k_perp * k_perp)
        inv = jax.lax.rsqrt(jnp.maximum(sq, EPS * EPS))
        u_new = k_perp * inv

        dr = jnp.sum(u_new * kq)
        di = jnp.sum(u_new * kqs)
        rowm = si4 == j
        ccq = jnp.concatenate([jnp.where(rowm, dr, crb),
                               jnp.where(rowm, di, cib)], axis=1)
        cqt = jax.lax.transpose(ccq, (1, 0))

        uo_ref[0, h] = jnp.where(rowm, mm(T4, u_new), A)

        Vb = v_ref[0, h]
        vq = vq_ref[0, h]
        V_new = g * jnp.where(rowm, mm(T4, vq), Vb)
        vo_ref[0, h] = V_new
        yq_ref[0, h] = mm(wrows(cqt[0:1], cqt[1:2]), V_new)


@functools.partial(jax.jit, static_argnames=())
def kernel(U, V, k, v, gamma, next_slot, filled):
    Uq = _to_q(U)
    Vq = _to_q(V)
    kq = _to_q_vec(k)
    vq = _to_q_vec(v)

    big_spec = pl.BlockSpec((1, H, QROWS, HALF), lambda b: (b, 0, 0, 0))
    vec_spec = pl.BlockSpec((1, H, 4, HALF), lambda b: (b, 0, 0, 0))
    smem_spec = pl.BlockSpec(memory_space=pltpu.SMEM)

    yq, U_newq, V_newq = pl.pallas_call(
        _sasaki_kernel,
        grid=(B,),
        in_specs=[smem_spec, smem_spec, big_spec, big_spec, vec_spec, vec_spec],
        out_specs=[vec_spec, big_spec, big_spec],
        out_shape=[
            jax.ShapeDtypeStruct((B, H, 4, HALF), jnp.float32),
            jax.ShapeDtypeStruct((B, H, QROWS, HALF), jnp.float32),
            jax.ShapeDtypeStruct((B, H, QROWS, HALF), jnp.float32),
        ],
    )(next_slot, gamma, Uq, Vq, kq, vq)

    y = _from_q_vec(yq)
    U_new = _from_q(U_newq)
    V_new = _from_q(V_newq)
    next_slot_new = (next_slot + 1) % RANK
    filled_new = jnp.minimum(filled + 1, jnp.full_like(filled, RANK))
    return (y, U_new, V_new, next_slot_new, filled_new)


# speedup vs baseline: 7.5094x; 3.2262x over previous
import functools

import jax
import jax.numpy as jnp
from jax.experimental import pallas as pl
from jax.experimental.pallas import tpu as pltpu

B, H, DIM, RANK = 64, 8, 256, 64
EPS = 1e-06
QROWS = 4 * RANK
HALF = DIM // 2


def _to_q(x):
    return (x.reshape(B, H, 2, HALF, RANK, 2)
             .transpose(0, 1, 4, 2, 5, 3)
             .reshape(B, H, QROWS, HALF))


def _from_q(xq):
    return (xq.reshape(B, H, RANK, 2, 2, HALF)
              .transpose(0, 1, 3, 5, 2, 4)
              .reshape(B, H, DIM, RANK, 2))


def _to_q_vec(x):
    return (x.reshape(B, H, 2, HALF, 2)
             .transpose(0, 1, 2, 4, 3)
             .reshape(B, H, 4, HALF))


def _from_q_vec(xq):
    return (xq.reshape(B, H, 2, 2, HALF)
              .transpose(0, 1, 2, 4, 3)
              .reshape(B, H, DIM, 2))


def _sasaki_kernel(ns_ref, gamma_ref, u_ref, v_ref, kq_ref, vq_ref,
                   yq_ref, uo_ref, vo_ref):
    bi = pl.program_id(0)
    f32 = jnp.float32

    si4 = jax.lax.broadcasted_iota(jnp.int32, (QROWS, 1), 0) // 4
    liq = jax.lax.broadcasted_iota(jnp.int32, (1, QROWS), 1) % 4
    e0 = (liq == 0).astype(f32)
    e1 = (liq == 1).astype(f32)
    e2 = (liq == 2).astype(f32)
    e3 = (liq == 3).astype(f32)
    t4r = jax.lax.broadcasted_iota(jnp.int32, (QROWS, 4), 0) % 4
    t4c = jax.lax.broadcasted_iota(jnp.int32, (QROWS, 4), 1)
    T4 = (t4r == t4c).astype(f32)
    s4r = jax.lax.broadcasted_iota(jnp.int32, (QROWS, QROWS), 0) // 4
    s4c = jax.lax.broadcasted_iota(jnp.int32, (QROWS, QROWS), 1) // 4
    S4 = (s4r == s4c).astype(f32)
    obr = jax.lax.broadcasted_iota(jnp.int32, (2 * H * HALF, 2 * H), 0) // HALF
    obc = jax.lax.broadcasted_iota(jnp.int32, (2 * H * HALF, 2 * H), 1)
    OBD = (obr == obc).astype(f32)

    def mm(a, b):
        return jax.lax.dot_general(a, b, (((1,), (0,)), ((), ())),
                                   preferred_element_type=f32)

    def wrows(crT, ciT):
        w0 = crT * e0 - ciT * e1
        w1 = ciT * e0 + crT * e1
        w2 = crT * e2 - ciT * e3
        w3 = ciT * e2 + crT * e3
        return jnp.concatenate([w0, w1, w2, w3], axis=0)

    A = [u_ref[0, h] for h in range(H)]
    kq = [kq_ref[0, h] for h in range(H)]
    kqs = [jnp.concatenate([k[1:2], -k[0:1], k[3:4], -k[2:3]], axis=0)
           for k in kq]
    j = [ns_ref[bi, h] for h in range(H)]
    rowm = [si4 == jh for jh in j]

    G1 = mm(T4, jnp.concatenate(kq, axis=1))
    G2 = mm(T4, jnp.concatenate(kqs, axis=1))
    TMP = jnp.concatenate(
        [jnp.concatenate(A, axis=1) * G1,
         jnp.concatenate(A, axis=1) * G2], axis=1)

    rdcat = mm(TMP, OBD)
    cc = mm(S4, rdcat)
    ct = jax.lax.transpose(cc, (1, 0))

    Wm = [wrows(ct[h:h + 1], ct[H + h:H + h + 1]) for h in range(H)]
    KP = [mm(Wm[h], A[h]) for h in range(H)]
    k_perp = [kq[h] - KP[h] for h in range(H)]
    inv = [jax.lax.rsqrt(jnp.maximum(jnp.sum(kp * kp), EPS * EPS))
           for kp in k_perp]
    u_new = [k_perp[h] * inv[h] for h in range(H)]

    dr = [jnp.sum(u_new[h] * kq[h]) for h in range(H)]
    di = [jnp.sum(u_new[h] * kqs[h]) for h in range(H)]
    ccq = jnp.concatenate(
        [jnp.where(rowm[h], dr[h], cc[:, h:h + 1]) for h in range(H)]
        + [jnp.where(rowm[h], di[h], cc[:, H + h:H + h + 1]) for h in range(H)],
        axis=1)
    cqt = jax.lax.transpose(ccq, (1, 0))

    u_t = mm(T4, jnp.concatenate(u_new, axis=1))
    v_t = mm(T4, jnp.concatenate(
        [vq_ref[0, h] for h in range(H)], axis=1))

    for h in range(H):
        g = jnp.clip(gamma_ref[bi, h], 0.0, 1.0)
        uo_ref[0, h] = jnp.where(rowm[h], u_t[:, h * HALF:(h + 1) * HALF],
                                 A[h])
        V_new = g * jnp.where(rowm[h], v_t[:, h * HALF:(h + 1) * HALF],
                              v_ref[0, h])
        vo_ref[0, h] = V_new
        yq_ref[0, h] = mm(wrows(cqt[h:h + 1], cqt[H + h:H + h + 1]), V_new)


@functools.partial(jax.jit, static_argnames=())
def kernel(U, V, k, v, gamma, next_slot, filled):
    Uq = _to_q(U)
    Vq = _to_q(V)
    kq = _to_q_vec(k)
    vq = _to_q_vec(v)

    big_spec = pl.BlockSpec((1, H, QROWS, HALF), lambda b: (b, 0, 0, 0))
    vec_spec = pl.BlockSpec((1, H, 4, HALF), lambda b: (b, 0, 0, 0))
    smem_spec = pl.BlockSpec(memory_space=pltpu.SMEM)

    yq, U_newq, V_newq = pl.pallas_call(
        _sasaki_kernel,
        grid=(B,),
        in_specs=[smem_spec, smem_spec, big_spec, big_spec, vec_spec, vec_spec],
        out_specs=[vec_spec, big_spec, big_spec],
        out_shape=[
            jax.ShapeDtypeStruct((B, H, 4, HALF), jnp.float32),
            jax.ShapeDtypeStruct((B, H, QROWS, HALF), jnp.float32),
            jax.ShapeDtypeStruct((B, H, QROWS, HALF), jnp.float32),
        ],
    )(next_slot, gamma, Uq, Vq, kq, vq)

    y = _from_q_vec(yq)
    U_new = _from_q(U_newq)
    V_new = _from_q(V_newq)
    next_slot_new = (next_slot + 1) % RANK
    filled_new = jnp.minimum(filled + 1, jnp.full_like(filled, RANK))
    return (y, U_new, V_new, next_slot_new, filled_new)


# speedup vs baseline: 7.5169x; 1.0010x over previous
import functools

import jax
import jax.numpy as jnp
from jax.experimental import pallas as pl
from jax.experimental.pallas import tpu as pltpu

B, H, DIM, RANK = 64, 8, 256, 64
EPS = 1e-06
QROWS = 4 * RANK
HALF = DIM // 2


def _to_q(x):
    return (x.reshape(B, H, 2, HALF, RANK, 2)
             .transpose(0, 1, 4, 2, 5, 3)
             .reshape(B, H, QROWS, HALF))


def _from_q(xq):
    return (xq.reshape(B, H, RANK, 2, 2, HALF)
              .transpose(0, 1, 3, 5, 2, 4)
              .reshape(B, H, DIM, RANK, 2))


def _to_q_vec(x):
    return (x.reshape(B, H, 2, HALF, 2)
             .transpose(0, 1, 2, 4, 3)
             .reshape(B, H, 4, HALF))


def _from_q_vec(xq):
    return (xq.reshape(B, H, 2, 2, HALF)
              .transpose(0, 1, 2, 4, 3)
              .reshape(B, H, DIM, 2))


def _sasaki_kernel(ns_ref, gamma_ref, u_ref, v_ref, kq_ref, vq_ref,
                   yq_ref, uo_ref, vo_ref):
    bi = pl.program_id(0)
    f32 = jnp.float32

    si4 = jax.lax.broadcasted_iota(jnp.int32, (QROWS, 1), 0) // 4
    liq = jax.lax.broadcasted_iota(jnp.int32, (1, QROWS), 1) % 4
    e0 = (liq == 0).astype(f32)
    e1 = (liq == 1).astype(f32)
    e2 = (liq == 2).astype(f32)
    e3 = (liq == 3).astype(f32)
    t4r = jax.lax.broadcasted_iota(jnp.int32, (QROWS, 4), 0) % 4
    t4c = jax.lax.broadcasted_iota(jnp.int32, (QROWS, 4), 1)
    T4 = (t4r == t4c).astype(f32)
    s4r = jax.lax.broadcasted_iota(jnp.int32, (QROWS, QROWS), 0) // 4
    s4c = jax.lax.broadcasted_iota(jnp.int32, (QROWS, QROWS), 1) // 4
    S4 = (s4r == s4c).astype(f32)
    obr = jax.lax.broadcasted_iota(jnp.int32, (2 * H * HALF, 2 * H), 0) // HALF
    obc = jax.lax.broadcasted_iota(jnp.int32, (2 * H * HALF, 2 * H), 1)
    OBD = (obr == obc).astype(f32)

    def mm(a, b, precision=None):
        return jax.lax.dot_general(a, b, (((1,), (0,)), ((), ())),
                                   preferred_element_type=f32,
                                   precision=precision)

    def wrows(crT, ciT):
        w0 = crT * e0 - ciT * e1
        w1 = ciT * e0 + crT * e1
        w2 = crT * e2 - ciT * e3
        w3 = ciT * e2 + crT * e3
        return jnp.concatenate([w0, w1, w2, w3], axis=0)

    A = [u_ref[0, h] for h in range(H)]
    kq = [kq_ref[0, h] for h in range(H)]
    kqs = [jnp.concatenate([k[1:2], -k[0:1], k[3:4], -k[2:3]], axis=0)
           for k in kq]
    j = [ns_ref[bi, h] for h in range(H)]
    rowm = [si4 == jh for jh in j]

    G1 = mm(T4, jnp.concatenate(kq, axis=1))
    G2 = mm(T4, jnp.concatenate(kqs, axis=1))
    TMP = jnp.concatenate(
        [jnp.concatenate(A, axis=1) * G1,
         jnp.concatenate(A, axis=1) * G2], axis=1)

    rdcat = mm(TMP, OBD)
    cc = mm(S4, rdcat)
    ct = jax.lax.transpose(cc, (1, 0))

    Wm = [wrows(ct[h:h + 1], ct[H + h:H + h + 1]) for h in range(H)]
    KP = [mm(Wm[h], A[h]) for h in range(H)]
    k_perp = [kq[h] - KP[h] for h in range(H)]
    inv = [jax.lax.rsqrt(jnp.maximum(jnp.sum(kp * kp), EPS * EPS))
           for kp in k_perp]
    u_new = [k_perp[h] * inv[h] for h in range(H)]

    dr = [jnp.sum(u_new[h] * kq[h]) for h in range(H)]
    di = [jnp.sum(u_new[h] * kqs[h]) for h in range(H)]
    ccq = jnp.concatenate(
        [jnp.where(rowm[h], dr[h], cc[:, h:h + 1]) for h in range(H)]
        + [jnp.where(rowm[h], di[h], cc[:, H + h:H + h + 1]) for h in range(H)],
        axis=1)
    cqt = jax.lax.transpose(ccq, (1, 0))

    u_t = mm(T4, jnp.concatenate(u_new, axis=1))
    v_t = mm(T4, jnp.concatenate(
        [vq_ref[0, h] for h in range(H)], axis=1))

    for h in range(H):
        g = jnp.clip(gamma_ref[bi, h], 0.0, 1.0)
        uo_ref[0, h] = jnp.where(rowm[h], u_t[:, h * HALF:(h + 1) * HALF],
                                 A[h])
        V_new = g * jnp.where(rowm[h], v_t[:, h * HALF:(h + 1) * HALF],
                              v_ref[0, h])
        vo_ref[0, h] = V_new
        yq_ref[0, h] = mm(wrows(cqt[h:h + 1], cqt[H + h:H + h + 1]), V_new)


@functools.partial(jax.jit, static_argnames=())
def kernel(U, V, k, v, gamma, next_slot, filled):
    Uq = _to_q(U)
    Vq = _to_q(V)
    kq = _to_q_vec(k)
    vq = _to_q_vec(v)

    big_spec = pl.BlockSpec((1, H, QROWS, HALF), lambda b: (b, 0, 0, 0))
    vec_spec = pl.BlockSpec((1, H, 4, HALF), lambda b: (b, 0, 0, 0))
    smem_spec = pl.BlockSpec(memory_space=pltpu.SMEM)

    yq, U_newq, V_newq = pl.pallas_call(
        _sasaki_kernel,
        grid=(B,),
        in_specs=[smem_spec, smem_spec, big_spec, big_spec, vec_spec, vec_spec],
        out_specs=[vec_spec, big_spec, big_spec],
        out_shape=[
            jax.ShapeDtypeStruct((B, H, 4, HALF), jnp.float32),
            jax.ShapeDtypeStruct((B, H, QROWS, HALF), jnp.float32),
            jax.ShapeDtypeStruct((B, H, QROWS, HALF), jnp.float32),
        ],
    )(next_slot, gamma, Uq, Vq, kq, vq)

    y = _from_q_vec(yq)
    U_new = _from_q(U_newq)
    V_new = _from_q(V_newq)
    next_slot_new = (next_slot + 1) % RANK
    filled_new = jnp.minimum(filled + 1, jnp.full_like(filled, RANK))
    return (y, U_new, V_new, next_slot_new, filled_new)


# speedup vs baseline: 8.8984x; 1.1838x over previous
import functools

import jax
import jax.numpy as jnp
from jax.experimental import pallas as pl
from jax.experimental.pallas import tpu as pltpu

B, H, DIM, RANK = 64, 8, 256, 64
EPS = 1e-06
QROWS = 4 * RANK
HALF = DIM // 2


def _to_q(x):
    return (x.reshape(B, H, 2, HALF, RANK, 2)
             .transpose(0, 1, 4, 2, 5, 3)
             .reshape(B, H, QROWS, HALF))


def _from_q(xq):
    return (xq.reshape(B, H, RANK, 2, 2, HALF)
              .transpose(0, 1, 3, 5, 2, 4)
              .reshape(B, H, DIM, RANK, 2))


def _to_q_vec(x):
    return (x.reshape(B, H, 2, HALF, 2)
             .transpose(0, 1, 2, 4, 3)
             .reshape(B, H, 4, HALF))


def _from_q_vec(xq):
    return (xq.reshape(B, H, 2, 2, HALF)
              .transpose(0, 1, 2, 4, 3)
              .reshape(B, H, DIM, 2))


def _sasaki_kernel(ns_ref, gamma_ref, u_ref, v_ref, kq_ref, vq_ref,
                   yq_ref, uo_ref, vo_ref):
    f32 = jnp.float32
    NB = 2
    NH = NB * H
    bi0 = pl.program_id(0) * NB

    si4 = jax.lax.broadcasted_iota(jnp.int32, (QROWS, 1), 0) // 4
    liq = jax.lax.broadcasted_iota(jnp.int32, (1, QROWS), 1) % 4
    e0 = (liq == 0).astype(f32)
    e1 = (liq == 1).astype(f32)
    e2 = (liq == 2).astype(f32)
    e3 = (liq == 3).astype(f32)
    t4r = jax.lax.broadcasted_iota(jnp.int32, (QROWS, 4), 0) % 4
    t4c = jax.lax.broadcasted_iota(jnp.int32, (QROWS, 4), 1)
    T4 = (t4r == t4c).astype(f32)
    s4r = jax.lax.broadcasted_iota(jnp.int32, (QROWS, QROWS), 0) // 4
    s4c = jax.lax.broadcasted_iota(jnp.int32, (QROWS, QROWS), 1) // 4
    S4 = (s4r == s4c).astype(f32)
    obr = jax.lax.broadcasted_iota(jnp.int32, (2 * NH * HALF, 2 * NH), 0) // HALF
    obc = jax.lax.broadcasted_iota(jnp.int32, (2 * NH * HALF, 2 * NH), 1)
    OBD = (obr == obc).astype(f32)

    def mm(a, b, precision=None):
        return jax.lax.dot_general(a, b, (((1,), (0,)), ((), ())),
                                   preferred_element_type=f32,
                                   precision=precision)

    def wrows(crT, ciT):
        w0 = crT * e0 - ciT * e1
        w1 = ciT * e0 + crT * e1
        w2 = crT * e2 - ciT * e3
        w3 = ciT * e2 + crT * e3
        return jnp.concatenate([w0, w1, w2, w3], axis=0)

    A = [u_ref[b, h] for b in range(NB) for h in range(H)]
    kq = [kq_ref[b, h] for b in range(NB) for h in range(H)]
    kqs = [jnp.concatenate([k[1:2], -k[0:1], k[3:4], -k[2:3]], axis=0)
           for k in kq]
    j = [ns_ref[bi0 + b, h] for b in range(NB) for h in range(H)]
    rowm = [si4 == jh for jh in j]

    G1 = mm(T4, jnp.concatenate(kq, axis=1))
    G2 = mm(T4, jnp.concatenate(kqs, axis=1))
    TMP = jnp.concatenate(
        [jnp.concatenate(A, axis=1) * G1,
         jnp.concatenate(A, axis=1) * G2], axis=1)

    rdcat = mm(TMP, OBD)
    cc = mm(S4, rdcat)
    ct = jax.lax.transpose(cc, (1, 0))

    Wm = [wrows(ct[n:n + 1], ct[NH + n:NH + n + 1]) for n in range(NH)]
    KP = [mm(Wm[n], A[n]) for n in range(NH)]
    k_perp = [kq[n] - KP[n] for n in range(NH)]
    inv = [jax.lax.rsqrt(jnp.maximum(jnp.sum(kp * kp), EPS * EPS))
           for kp in k_perp]
    u_new = [k_perp[n] * inv[n] for n in range(NH)]

    dr = [jnp.sum(u_new[n] * kq[n]) for n in range(NH)]
    di = [jnp.sum(u_new[n] * kqs[n]) for n in range(NH)]
    ccq = jnp.concatenate(
        [jnp.where(rowm[n], dr[n], cc[:, n:n + 1]) for n in range(NH)]
        + [jnp.where(rowm[n], di[n], cc[:, NH + n:NH + n + 1]) for n in range(NH)],
        axis=1)
    cqt = jax.lax.transpose(ccq, (1, 0))

    u_t = mm(T4, jnp.concatenate(u_new, axis=1))
    v_t = mm(T4, jnp.concatenate(
        [vq_ref[b, h] for b in range(NB) for h in range(H)], axis=1))

    for b in range(NB):
        for h in range(H):
            n = b * H + h
            g = jnp.clip(gamma_ref[bi0 + b, h], 0.0, 1.0)
            uo_ref[b, h] = jnp.where(rowm[n], u_t[:, n * HALF:(n + 1) * HALF],
                                     A[n])
            V_new = g * jnp.where(rowm[n], v_t[:, n * HALF:(n + 1) * HALF],
                                  v_ref[b, h])
            vo_ref[b, h] = V_new
            yq_ref[b, h] = mm(wrows(cqt[n:n + 1], cqt[NH + n:NH + n + 1]),
                              V_new)


@functools.partial(jax.jit, static_argnames=())
def kernel(U, V, k, v, gamma, next_slot, filled):
    Uq = _to_q(U)
    Vq = _to_q(V)
    kq = _to_q_vec(k)
    vq = _to_q_vec(v)

    big_spec = pl.BlockSpec((2, H, QROWS, HALF), lambda b: (b, 0, 0, 0))
    vec_spec = pl.BlockSpec((2, H, 4, HALF), lambda b: (b, 0, 0, 0))
    smem_spec = pl.BlockSpec(memory_space=pltpu.SMEM)

    yq, U_newq, V_newq = pl.pallas_call(
        _sasaki_kernel,
        grid=(B // 2,),
        in_specs=[smem_spec, smem_spec, big_spec, big_spec, vec_spec, vec_spec],
        out_specs=[vec_spec, big_spec, big_spec],
        out_shape=[
            jax.ShapeDtypeStruct((B, H, 4, HALF), jnp.float32),
            jax.ShapeDtypeStruct((B, H, QROWS, HALF), jnp.float32),
            jax.ShapeDtypeStruct((B, H, QROWS, HALF), jnp.float32),
        ],
    )(next_slot, gamma, Uq, Vq, kq, vq)

    y = _from_q_vec(yq)
    U_new = _from_q(U_newq)
    V_new = _from_q(V_newq)
    next_slot_new = (next_slot + 1) % RANK
    filled_new = jnp.minimum(filled + 1, jnp.full_like(filled, RANK))
    return (y, U_new, V_new, next_slot_new, filled_new)


# speedup vs baseline: 9.2621x; 1.0409x over previous
import functools

import jax
import jax.numpy as jnp
from jax.experimental import pallas as pl
from jax.experimental.pallas import tpu as pltpu

B, H, DIM, RANK = 64, 8, 256, 64
EPS = 1e-06
QROWS = 4 * RANK
HALF = DIM // 2


def _to_q(x):
    return (x.reshape(B, H, 2, HALF, RANK, 2)
             .transpose(0, 1, 4, 2, 5, 3)
             .reshape(B, H, QROWS, HALF))


def _from_q(xq):
    return (xq.reshape(B, H, RANK, 2, 2, HALF)
              .transpose(0, 1, 3, 5, 2, 4)
              .reshape(B, H, DIM, RANK, 2))


def _to_q_vec(x):
    return (x.reshape(B, H, 2, HALF, 2)
             .transpose(0, 1, 2, 4, 3)
             .reshape(B, H, 4, HALF))


def _from_q_vec(xq):
    return (xq.reshape(B, H, 2, 2, HALF)
              .transpose(0, 1, 2, 4, 3)
              .reshape(B, H, DIM, 2))


def _sasaki_kernel(ns_ref, gamma_ref, u_ref, v_ref, kq_ref, vq_ref,
                   yq_ref, uo_ref, vo_ref):
    f32 = jnp.float32
    NB = 4
    NH = NB * H
    bi0 = pl.program_id(0) * NB

    si4 = jax.lax.broadcasted_iota(jnp.int32, (QROWS, 1), 0) // 4
    liq = jax.lax.broadcasted_iota(jnp.int32, (1, QROWS), 1) % 4
    e0 = (liq == 0).astype(f32)
    e1 = (liq == 1).astype(f32)
    e2 = (liq == 2).astype(f32)
    e3 = (liq == 3).astype(f32)
    t4r = jax.lax.broadcasted_iota(jnp.int32, (QROWS, 4), 0) % 4
    t4c = jax.lax.broadcasted_iota(jnp.int32, (QROWS, 4), 1)
    T4 = (t4r == t4c).astype(f32)
    s4r = jax.lax.broadcasted_iota(jnp.int32, (QROWS, QROWS), 0) // 4
    s4c = jax.lax.broadcasted_iota(jnp.int32, (QROWS, QROWS), 1) // 4
    S4 = (s4r == s4c).astype(f32)
    obr = jax.lax.broadcasted_iota(jnp.int32, (2 * NH * HALF, 2 * NH), 0) // HALF
    obc = jax.lax.broadcasted_iota(jnp.int32, (2 * NH * HALF, 2 * NH), 1)
    OBD = (obr == obc).astype(f32)

    def mm(a, b, precision=None):
        return jax.lax.dot_general(a, b, (((1,), (0,)), ((), ())),
                                   preferred_element_type=f32,
                                   precision=precision)

    def wrows(crT, ciT):
        w0 = crT * e0 - ciT * e1
        w1 = ciT * e0 + crT * e1
        w2 = crT * e2 - ciT * e3
        w3 = ciT * e2 + crT * e3
        return jnp.concatenate([w0, w1, w2, w3], axis=0)

    A = [u_ref[b, h] for b in range(NB) for h in range(H)]
    kq = [kq_ref[b, h] for b in range(NB) for h in range(H)]
    kqs = [jnp.concatenate([k[1:2], -k[0:1], k[3:4], -k[2:3]], axis=0)
           for k in kq]
    j = [ns_ref[bi0 + b, h] for b in range(NB) for h in range(H)]
    rowm = [si4 == jh for jh in j]

    G1 = mm(T4, jnp.concatenate(kq, axis=1))
    G2 = mm(T4, jnp.concatenate(kqs, axis=1))
    TMP = jnp.concatenate(
        [jnp.concatenate(A, axis=1) * G1,
         jnp.concatenate(A, axis=1) * G2], axis=1)

    rdcat = mm(TMP, OBD)
    cc = mm(S4, rdcat)
    ct = jax.lax.transpose(cc, (1, 0))

    Wm = [wrows(ct[n:n + 1], ct[NH + n:NH + n + 1]) for n in range(NH)]
    KP = [mm(Wm[n], A[n]) for n in range(NH)]
    k_perp = [kq[n] - KP[n] for n in range(NH)]
    inv = [jax.lax.rsqrt(jnp.maximum(jnp.sum(kp * kp), EPS * EPS))
           for kp in k_perp]
    u_new = [k_perp[n] * inv[n] for n in range(NH)]

    dr = [jnp.sum(u_new[n] * kq[n]) for n in range(NH)]
    di = [jnp.sum(u_new[n] * kqs[n]) for n in range(NH)]
    ccq = jnp.concatenate(
        [jnp.where(rowm[n], dr[n], cc[:, n:n + 1]) for n in range(NH)]
        + [jnp.where(rowm[n], di[n], cc[:, NH + n:NH + n + 1]) for n in range(NH)],
        axis=1)
    cqt = jax.lax.transpose(ccq, (1, 0))

    u_t = mm(T4, jnp.concatenate(u_new, axis=1))
    v_t = mm(T4, jnp.concatenate(
        [vq_ref[b, h] for b in range(NB) for h in range(H)], axis=1))

    for b in range(NB):
        for h in range(H):
            n = b * H + h
            g = jnp.clip(gamma_ref[bi0 + b, h], 0.0, 1.0)
            uo_ref[b, h] = jnp.where(rowm[n], u_t[:, n * HALF:(n + 1) * HALF],
                                     A[n])
            V_new = g * jnp.where(rowm[n], v_t[:, n * HALF:(n + 1) * HALF],
                                  v_ref[b, h])
            vo_ref[b, h] = V_new
            yq_ref[b, h] = mm(wrows(cqt[n:n + 1], cqt[NH + n:NH + n + 1]),
                              V_new)


@functools.partial(jax.jit, static_argnames=())
def kernel(U, V, k, v, gamma, next_slot, filled):
    Uq = _to_q(U)
    Vq = _to_q(V)
    kq = _to_q_vec(k)
    vq = _to_q_vec(v)

    big_spec = pl.BlockSpec((4, H, QROWS, HALF), lambda b: (b, 0, 0, 0))
    vec_spec = pl.BlockSpec((4, H, 4, HALF), lambda b: (b, 0, 0, 0))
    smem_spec = pl.BlockSpec(memory_space=pltpu.SMEM)

    yq, U_newq, V_newq = pl.pallas_call(
        _sasaki_kernel,
        grid=(B // 4,),
        in_specs=[smem_spec, smem_spec, big_spec, big_spec, vec_spec, vec_spec],
        out_specs=[vec_spec, big_spec, big_spec],
        out_shape=[
            jax.ShapeDtypeStruct((B, H, 4, HALF), jnp.float32),
            jax.ShapeDtypeStruct((B, H, QROWS, HALF), jnp.float32),
            jax.ShapeDtypeStruct((B, H, QROWS, HALF), jnp.float32),
        ],
    )(next_slot, gamma, Uq, Vq, kq, vq)

    y = _from_q_vec(yq)
    U_new = _from_q(U_newq)
    V_new = _from_q(V_newq)
    next_slot_new = (next_slot + 1) % RANK
    filled_new = jnp.minimum(filled + 1, jnp.full_like(filled, RANK))
    return (y, U_new, V_new, next_slot_new, filled_new)
